# Initial kernel scaffold; baseline (speedup 1.0000x reference)
import jax
import jax.numpy as jnp
from jax.experimental import pallas as pl


def _body(emb_ref, out_ref):
    out_ref[...] = jnp.broadcast_to(emb_ref[:, None, :], out_ref.shape)


def kernel(x, pos_embedding):
    seq_len, batch = x.shape
    max_len, embed_dim = pos_embedding.shape
    blk = 512
    out = pl.pallas_call(
        _body,
        grid=(seq_len // blk,),
        in_specs=[pl.BlockSpec((blk, embed_dim), lambda i: (i, 0))],
        out_specs=pl.BlockSpec((blk, batch, embed_dim), lambda i: (i, 0, 0)),
        out_shape=jax.ShapeDtypeStruct(
            (seq_len, batch, embed_dim), pos_embedding.dtype
        ),
    )(pos_embedding)
    return out


# speedup vs baseline: 4.2712x; 4.2712x over previous
import functools

import jax
import jax.numpy as jnp
from jax import lax
from jax.experimental import pallas as pl
from jax.experimental.pallas import tpu as pltpu
from jax.experimental.pallas import tpu_sc as plsc

_NC = 2
_NS = 16
_NW = _NC * _NS


@functools.lru_cache(maxsize=None)
def _make_sc(seq_len, batch, embed_dim, dtype):
    rows_per_w = seq_len // _NW
    chunk = min(rows_per_w, 64)
    n_chunks = rows_per_w // chunk
    n_buf = 2
    mesh = plsc.VectorSubcoreMesh(core_axis_name="c", subcore_axis_name="s")

    @functools.partial(
        pl.kernel,
        mesh=mesh,
        out_type=jax.ShapeDtypeStruct((seq_len, batch, embed_dim), dtype),
        scratch_types=(
            [pltpu.VMEM((chunk, 1, embed_dim), dtype) for _ in range(n_buf)]
            + [pltpu.SemaphoreType.DMA for _ in range(n_buf)]
            + [pltpu.SemaphoreType.DMA for _ in range(n_buf)]
        ),
    )
    def k(table_hbm, out_hbm, *scratch):
        bufs = scratch[:n_buf]
        lsem = scratch[n_buf : 2 * n_buf]
        wsem = scratch[2 * n_buf : 3 * n_buf]
        wid = lax.axis_index("s") * _NC + lax.axis_index("c")
        base = wid * rows_per_w

        def load(c):
            s0 = base + c * chunk
            d = pltpu.make_async_copy(
                table_hbm.at[pl.ds(s0, chunk)], bufs[c % n_buf], lsem[c % n_buf]
            )
            d.start()
            return d

        def writes(c):
            s0 = base + c * chunk
            ds = []
            for b in range(batch):
                d = pltpu.make_async_copy(
                    bufs[c % n_buf],
                    out_hbm.at[pl.ds(s0, chunk), pl.ds(b, 1)],
                    wsem[c % n_buf],
                )
                d.start()
                ds.append(d)
            return ds

        pending_w = [None] * n_buf
        ld = load(0)
        for c in range(n_chunks):
            cur = c % n_buf
            if c + 1 < n_chunks:
                nxt = (c + 1) % n_buf
                if pending_w[nxt] is not None:
                    for d in pending_w[nxt]:
                        d.wait()
                    pending_w[nxt] = None
                ld_next = load(c + 1)
            ld.wait()
            pending_w[cur] = writes(c)
            if c + 1 < n_chunks:
                ld = ld_next
        for ds in pending_w:
            if ds is not None:
                for d in ds:
                    d.wait()

    return k


def kernel(x, pos_embedding):
    seq_len, batch = x.shape
    max_len, embed_dim = pos_embedding.shape
    k = _make_sc(seq_len, batch, embed_dim, pos_embedding.dtype)
    table3 = pos_embedding[:seq_len].reshape(seq_len, 1, embed_dim)
    return k(table3)


# speedup vs baseline: 5.5069x; 1.2893x over previous
import functools

import jax
import jax.numpy as jnp
from jax import lax
from jax.experimental import pallas as pl
from jax.experimental.pallas import tpu as pltpu
from jax.experimental.pallas import tpu_sc as plsc

_NC = 2
_NS = 16
_NW = _NC * _NS


@functools.lru_cache(maxsize=None)
def _make_sc(seq_len, batch, embed_dim, dtype):
    rows_per_w = seq_len // _NW
    chunk = min(rows_per_w, 64)
    n_chunks = rows_per_w // chunk
    n_buf = 2
    mesh = plsc.VectorSubcoreMesh(core_axis_name="c", subcore_axis_name="s")

    @functools.partial(
        pl.kernel,
        mesh=mesh,
        out_type=jax.ShapeDtypeStruct((seq_len, batch, embed_dim), dtype),
        scratch_types=(
            [pltpu.VMEM((chunk, embed_dim), dtype) for _ in range(n_buf)]
            + [pltpu.SemaphoreType.DMA for _ in range(n_buf)]
            + [pltpu.SemaphoreType.DMA for _ in range(n_buf)]
        ),
    )
    def k(table_hbm, out_hbm, *scratch):
        bufs = scratch[:n_buf]
        lsem = scratch[n_buf : 2 * n_buf]
        wsem = scratch[2 * n_buf : 3 * n_buf]
        wid = lax.axis_index("s") * _NC + lax.axis_index("c")
        base = wid * rows_per_w

        def load(c):
            s0 = base + c * chunk
            d = pltpu.make_async_copy(
                table_hbm.at[pl.ds(s0, chunk)], bufs[c % n_buf], lsem[c % n_buf]
            )
            d.start()
            return d

        def writes(c):
            s0 = base + c * chunk
            ds = []
            for b in range(batch):
                d = pltpu.make_async_copy(
                    bufs[c % n_buf],
                    out_hbm.at[pl.ds(s0, chunk), b],
                    wsem[c % n_buf],
                )
                d.start()
                ds.append(d)
            return ds

        pending_w = [None] * n_buf
        ld = load(0)
        for c in range(n_chunks):
            cur = c % n_buf
            if c + 1 < n_chunks:
                nxt = (c + 1) % n_buf
                if pending_w[nxt] is not None:
                    for d in pending_w[nxt]:
                        d.wait()
                    pending_w[nxt] = None
                ld_next = load(c + 1)
            ld.wait()
            pending_w[cur] = writes(c)
            if c + 1 < n_chunks:
                ld = ld_next
        for ds in pending_w:
            if ds is not None:
                for d in ds:
                    d.wait()

    return k


def kernel(x, pos_embedding):
    seq_len, batch = x.shape
    max_len, embed_dim = pos_embedding.shape
    k = _make_sc(seq_len, batch, embed_dim, pos_embedding.dtype)
    return k(pos_embedding)
